# baseline (speedup 1.0000x reference)
import functools

import jax
import jax.numpy as jnp
from jax import lax
from jax.experimental import pallas as pl
from jax.experimental.pallas import tpu as pltpu
from jax.experimental.pallas import tpu_sc as plsc

V = 1000000
D = 64
NC = 2
NS = 16
NW = NC * NS
PITCH = 128
NG = 3
NO = 2

_MESH = dict(core_axis_name="c", subcore_axis_name="s")


def _lookup_body(ids_t, scratch, out_t, idx_v, g_v, ot_v, *sems):
    hist = ids_t.shape[0]
    bpw = ids_t.shape[1] // NW
    gsems, ssems = sems[:2 * NG], sems[2 * NG:2 * NG + NO]
    wid = lax.axis_index("s") * NC + lax.axis_index("c")
    base = wid * bpw
    pltpu.sync_copy(ids_t.at[:, pl.ds(base, bpw)], idx_v)

    def g_half(h, b, half):
        return pltpu.make_async_copy(
            scratch.at[idx_v.at[h, pl.ds(64 * half, 64)]],
            g_v.at[b, pl.ds(64 * half, 64)], gsems[2 * b + half])

    def g_start(h, b):
        g_half(h, b, 0).start()
        g_half(h, b, 1).start()

    def s_start(h, b):
        pltpu.make_async_copy(
            ot_v.at[b, :, pl.ds(0, bpw)], out_t.at[h, :, pl.ds(base, bpw)],
            ssems[b]).start()

    def s_wait(h, b):
        pltpu.make_async_copy(
            ot_v.at[b, :, pl.ds(0, bpw)], out_t.at[h, :, pl.ds(base, bpw)],
            ssems[b]).wait()

    for b in range(NG):
        g_start(b, b)
    rows = [lax.iota(jnp.int32, 16) + 16 * k for k in range(8)]

    def step(h, bg, bo, first):
        if not first:
            s_wait(h - NO, bo)
        for half in range(2):
            g_half(h, bg, half).wait()

            @plsc.parallel_loop(0, 64, unroll=8,
                                carry=jnp.full((16,), 64 * half, jnp.int32))
            def _(b, colv):
                bb = 64 * half + b
                for k in range(4):
                    plsc.store_scatter(
                        ot_v.at[bo], [rows[k], colv],
                        g_v[bg, bb, pl.ds(16 * k, 16)])
                return colv + 1
        s_start(h, bo)

        @pl.when(h + NG < hist)
        def _():
            g_start(h + NG, bg)

    period = NG * NO
    nmain = (hist - 2) // period

    def body(i, carry):
        for j in range(period):
            h = i * period + j
            step(h, j % NG, j % NO, False)
        return carry

    for j in range(period):
        step(j, j % NG, j % NO, j < NO)

    def body2(i, carry):
        for j in range(period):
            h = (i + 1) * period + j
            step(h, j % NG, j % NO, False)
        return carry

    lax.fori_loop(0, nmain - 1, body2, 0)
    for h in range(nmain * period, hist):
        step(h, h % NG, h % NO, False)
    s_wait(hist - 2, (hist - 2) % NO)
    s_wait(hist - 1, (hist - 1) % NO)


@jax.jit
def kernel(token_ids, embeddings):
    bsz, hist = token_ids.shape
    mesh = plsc.VectorSubcoreMesh(num_cores=NC, num_subcores=NS, **_MESH)
    scratch = jnp.pad(embeddings, ((0, 0), (0, PITCH - D)))
    lookup = pl.kernel(
        _lookup_body,
        out_type=jax.ShapeDtypeStruct((hist, D, bsz), jnp.float32),
        mesh=mesh,
        scratch_types=[
            pltpu.VMEM((hist, bsz // NW), jnp.int32),
            pltpu.VMEM((NG, bsz // NW, PITCH), jnp.float32),
            pltpu.VMEM((NO, D, bsz // NW + 1), jnp.float32),
        ] + [pltpu.SemaphoreType.DMA] * (2 * NG + NO),
        compiler_params=pltpu.CompilerParams(
            use_tc_tiling_on_sc=True, needs_layout_passes=False),
    )
    out_t = lookup(jnp.transpose(token_ids).astype(jnp.int32), scratch)
    return jnp.transpose(out_t, (2, 0, 1))


# speedup vs baseline: 1.0537x; 1.0537x over previous
import functools

import jax
import jax.numpy as jnp
from jax import lax
from jax.experimental import pallas as pl
from jax.experimental.pallas import tpu as pltpu
from jax.experimental.pallas import tpu_sc as plsc

V = 1000000
D = 64
NC = 2
NS = 16
NW = NC * NS
PITCH = 128
NG = 3
NO = 2

_MESH = dict(core_axis_name="c", subcore_axis_name="s")


def _lookup_body(ids_t, scratch, out_t, idx_v, g_v, ot_v, *sems):
    hist = ids_t.shape[0]
    bpw = ids_t.shape[1] // NW
    gsems, ssems = sems[:NG], sems[NG:NG + NO]
    wid = lax.axis_index("s") * NC + lax.axis_index("c")
    base = wid * bpw
    pltpu.sync_copy(ids_t.at[:, pl.ds(base, bpw)], idx_v)

    def g_start(h, b):
        pltpu.make_async_copy(
            scratch.at[idx_v.at[h]], g_v.at[b], gsems[b]).start()

    def g_wait(h, b):
        pltpu.make_async_copy(
            scratch.at[idx_v.at[h]], g_v.at[b], gsems[b]).wait()

    def s_start(h, b):
        pltpu.make_async_copy(
            ot_v.at[b], out_t.at[h, :, pl.ds(base, bpw)], ssems[b]).start()

    def s_wait(h, b):
        pltpu.make_async_copy(
            ot_v.at[b], out_t.at[h, :, pl.ds(base, bpw)], ssems[b]).wait()

    for b in range(NG):
        g_start(b, b)
    rows = [lax.iota(jnp.int32, 16) + 16 * k for k in range(8)]

    def step(h, bg, bo, first):
        g_wait(h, bg)
        if not first:
            s_wait(h - NO, bo)

        @plsc.parallel_loop(0, D, unroll=8)
        def _(f):
            cols = jnp.full((16,), f, jnp.int32)
            for k in range(8):
                ot_v[bo, f, pl.ds(16 * k, 16)] = plsc.load_gather(
                    g_v.at[bg], [rows[k], cols])
        s_start(h, bo)

        @pl.when(h + NG < hist)
        def _():
            g_start(h + NG, bg)

    period = NG * NO
    nmain = (hist - 2) // period

    def body(i, carry):
        for j in range(period):
            h = i * period + j
            step(h, j % NG, j % NO, False)
        return carry

    for j in range(period):
        step(j, j % NG, j % NO, j < NO)

    def body2(i, carry):
        for j in range(period):
            h = (i + 1) * period + j
            step(h, j % NG, j % NO, False)
        return carry

    lax.fori_loop(0, nmain - 1, body2, 0)
    for h in range(nmain * period, hist):
        step(h, h % NG, h % NO, False)
    s_wait(hist - 2, (hist - 2) % NO)
    s_wait(hist - 1, (hist - 1) % NO)


@jax.jit
def kernel(token_ids, embeddings):
    bsz, hist = token_ids.shape
    mesh = plsc.VectorSubcoreMesh(num_cores=NC, num_subcores=NS, **_MESH)
    scratch = jnp.pad(embeddings, ((0, 0), (0, PITCH - D)))
    lookup = pl.kernel(
        _lookup_body,
        out_type=jax.ShapeDtypeStruct((hist, D, bsz), jnp.float32),
        mesh=mesh,
        scratch_types=[
            pltpu.VMEM((hist, bsz // NW), jnp.int32),
            pltpu.VMEM((NG, bsz // NW, PITCH), jnp.float32),
            pltpu.VMEM((NO, D, bsz // NW), jnp.float32),
        ] + [pltpu.SemaphoreType.DMA] * (NG + NO),
        compiler_params=pltpu.CompilerParams(
            use_tc_tiling_on_sc=True, needs_layout_passes=False),
    )
    out_t = lookup(jnp.transpose(token_ids).astype(jnp.int32), scratch)
    return jnp.transpose(out_t, (2, 0, 1))


# speedup vs baseline: 1.0574x; 1.0035x over previous
import functools

import jax
import jax.numpy as jnp
from jax import lax
from jax.experimental import pallas as pl
from jax.experimental.pallas import tpu as pltpu
from jax.experimental.pallas import tpu_sc as plsc

V = 1000000
D = 64
NC = 2
NS = 16
NW = NC * NS
PITCH = 128
NG = 3
NO = 2

_MESH = dict(core_axis_name="c", subcore_axis_name="s")


def _lookup_body(ids_t, scratch, out_t, idx_v, g_v, ot_v, *sems):
    hist = ids_t.shape[0]
    bpw = ids_t.shape[1] // NW
    gsems, ssems = sems[:NG], sems[NG:NG + NO]
    wid = lax.axis_index("s") * NC + lax.axis_index("c")
    base = wid * bpw
    pltpu.sync_copy(ids_t.at[:, pl.ds(base, bpw)], idx_v)

    def g_start(h, b):
        pltpu.make_async_copy(
            scratch.at[idx_v.at[h]], g_v.at[b], gsems[b]).start()

    def g_wait(h, b):
        pltpu.make_async_copy(
            scratch.at[idx_v.at[h]], g_v.at[b], gsems[b]).wait()

    def s_start(h, b):
        pltpu.make_async_copy(
            ot_v.at[b], out_t.at[h, :, pl.ds(base, bpw)], ssems[b]).start()

    def s_wait(h, b):
        pltpu.make_async_copy(
            ot_v.at[b], out_t.at[h, :, pl.ds(base, bpw)], ssems[b]).wait()

    for b in range(NG):
        g_start(b, b)
    rows = [lax.iota(jnp.int32, 16) + 16 * k for k in range(8)]

    def step(h, bg, bo, first):
        g_wait(h, bg)
        if not first:
            s_wait(h - NO, bo)

        @plsc.parallel_loop(0, D, unroll=4)
        def _(f):
            cols = jnp.full((16,), f, jnp.int32)
            for k in range(8):
                ot_v[bo, f, pl.ds(16 * k, 16)] = plsc.load_gather(
                    g_v.at[bg], [rows[k], cols])
        s_start(h, bo)

        @pl.when(h + NG < hist)
        def _():
            g_start(h + NG, bg)

    period = NG * NO
    nmain = (hist - 2) // period

    def body(i, carry):
        for j in range(period):
            h = i * period + j
            step(h, j % NG, j % NO, False)
        return carry

    for j in range(period):
        step(j, j % NG, j % NO, j < NO)

    def body2(i, carry):
        for j in range(period):
            h = (i + 1) * period + j
            step(h, j % NG, j % NO, False)
        return carry

    lax.fori_loop(0, nmain - 1, body2, 0)
    for h in range(nmain * period, hist):
        step(h, h % NG, h % NO, False)
    s_wait(hist - 2, (hist - 2) % NO)
    s_wait(hist - 1, (hist - 1) % NO)


@jax.jit
def kernel(token_ids, embeddings):
    bsz, hist = token_ids.shape
    mesh = plsc.VectorSubcoreMesh(num_cores=NC, num_subcores=NS, **_MESH)
    scratch = jnp.pad(embeddings, ((0, 0), (0, PITCH - D)))
    lookup = pl.kernel(
        _lookup_body,
        out_type=jax.ShapeDtypeStruct((hist, D, bsz), jnp.float32),
        mesh=mesh,
        scratch_types=[
            pltpu.VMEM((hist, bsz // NW), jnp.int32),
            pltpu.VMEM((NG, bsz // NW, PITCH), jnp.float32),
            pltpu.VMEM((NO, D, bsz // NW), jnp.float32),
        ] + [pltpu.SemaphoreType.DMA] * (NG + NO),
        compiler_params=pltpu.CompilerParams(
            use_tc_tiling_on_sc=True, needs_layout_passes=False),
    )
    out_t = lookup(jnp.transpose(token_ids).astype(jnp.int32), scratch)
    return jnp.transpose(out_t, (2, 0, 1))
